# baseline (speedup 1.0000x reference)
import functools
import math

import jax
import jax.numpy as jnp
from jax import lax
from jax.experimental import pallas as pl
from jax.experimental.pallas import tpu as pltpu
from jax.experimental.pallas import tpu_sc as plsc

B, S, D = 1, 2048, 768
H, KV, DH = 12, 4, 64
E, TOPK, FF = 8, 2, 768
EPS = 1e-5
THETA = 150000.0
ALPHA = 1.702
LIMIT = 7.0

BS1 = 256
BQ = 256
BT = 128


def _rms_in(x, w):
    v = jnp.mean(jnp.square(x), axis=-1, keepdims=True)
    return x * jax.lax.rsqrt(v + EPS) * w


def _k1_body(h_ref, ln1_ref, wq_ref, bq_ref, wk_ref, bk_ref, wv_ref, bv_ref,
             q_ref, k_ref, v_ref):
    x = _rms_in(h_ref[...], ln1_ref[...])
    q_ref[...] = jnp.dot(x, wq_ref[...], preferred_element_type=jnp.float32) + bq_ref[...]
    k_ref[...] = jnp.dot(x, wk_ref[...], preferred_element_type=jnp.float32) + bk_ref[...]
    v_ref[...] = jnp.dot(x, wv_ref[...], preferred_element_type=jnp.float32) + bv_ref[...]


def _k1(h, ln1_w, wq, bq, wk, bk, wv, bv):
    n = S // BS1
    return pl.pallas_call(
        _k1_body,
        grid=(n,),
        in_specs=[
            pl.BlockSpec((BS1, D), lambda i: (i, 0)),
            pl.BlockSpec((1, D), lambda i: (0, 0)),
            pl.BlockSpec((D, H * DH), lambda i: (0, 0)),
            pl.BlockSpec((1, H * DH), lambda i: (0, 0)),
            pl.BlockSpec((D, KV * DH), lambda i: (0, 0)),
            pl.BlockSpec((1, KV * DH), lambda i: (0, 0)),
            pl.BlockSpec((D, KV * DH), lambda i: (0, 0)),
            pl.BlockSpec((1, KV * DH), lambda i: (0, 0)),
        ],
        out_specs=[
            pl.BlockSpec((BS1, H * DH), lambda i: (i, 0)),
            pl.BlockSpec((BS1, KV * DH), lambda i: (i, 0)),
            pl.BlockSpec((BS1, KV * DH), lambda i: (i, 0)),
        ],
        out_shape=[
            jax.ShapeDtypeStruct((S, H * DH), jnp.float32),
            jax.ShapeDtypeStruct((S, KV * DH), jnp.float32),
            jax.ShapeDtypeStruct((S, KV * DH), jnp.float32),
        ],
    )(h, ln1_w[None, :], wq, bq[None, :], wk, bk[None, :], wv, bv[None, :])


def _rope(x, cos, sin):
    x1 = x[:, :DH // 2]
    x2 = x[:, DH // 2:]
    return jnp.concatenate([x1 * cos - x2 * sin, x2 * cos + x1 * sin], axis=1)


def _k2_body(q_ref, k_ref, v_ref, cq_ref, sq_ref, ck_ref, sk_ref, sink_ref,
             o_ref):
    h = pl.program_id(0)
    iq = pl.program_id(1)
    q = _rope(q_ref[0], cq_ref[...], sq_ref[...]) * (1.0 / math.sqrt(DH))
    row = iq * BQ + jax.lax.broadcasted_iota(jnp.int32, (BQ, BQ), 0)

    def chunk(j, carry):
        m, l, acc = carry
        kc = _rope(k_ref[0, pl.ds(j * BQ, BQ), :],
                   ck_ref[pl.ds(j * BQ, BQ), :], sk_ref[pl.ds(j * BQ, BQ), :])
        lc = jax.lax.dot_general(q, kc, (((1,), (1,)), ((), ())),
                                 preferred_element_type=jnp.float32)
        col = j * BQ + jax.lax.broadcasted_iota(jnp.int32, (BQ, BQ), 1)
        lc = jnp.where(col <= row, lc, -1e9)
        mj = jnp.maximum(m, jnp.max(lc, axis=1, keepdims=True))
        p = jnp.exp(lc - mj)
        corr = jnp.exp(m - mj)
        l = l * corr + jnp.sum(p, axis=1, keepdims=True)
        acc = acc * corr + jnp.dot(p, v_ref[0, pl.ds(j * BQ, BQ), :],
                                   preferred_element_type=jnp.float32)
        return mj, l, acc

    m0 = jnp.full((BQ, 1), -1e30, jnp.float32)
    l0 = jnp.zeros((BQ, 1), jnp.float32)
    a0 = jnp.zeros((BQ, DH), jnp.float32)
    m, l, acc = lax.fori_loop(0, iq + 1, chunk, (m0, l0, a0))
    sink = sink_ref[h]
    mf = jnp.maximum(m, sink)
    corr = jnp.exp(m - mf)
    denom = l * corr + jnp.exp(sink - mf)
    o_ref[0] = acc * corr / denom


def _k2(q, k, v, cos, sin, sinks):
    rep = H // KV
    nq = S // BQ
    return pl.pallas_call(
        _k2_body,
        grid=(H, nq),
        in_specs=[
            pl.BlockSpec((1, BQ, DH), lambda h, i: (h, i, 0)),
            pl.BlockSpec((1, S, DH), lambda h, i: (h // rep, 0, 0)),
            pl.BlockSpec((1, S, DH), lambda h, i: (h // rep, 0, 0)),
            pl.BlockSpec((BQ, DH // 2), lambda h, i: (i, 0)),
            pl.BlockSpec((BQ, DH // 2), lambda h, i: (i, 0)),
            pl.BlockSpec((S, DH // 2), lambda h, i: (0, 0)),
            pl.BlockSpec((S, DH // 2), lambda h, i: (0, 0)),
            pl.BlockSpec(memory_space=pltpu.SMEM),
        ],
        out_specs=pl.BlockSpec((1, BQ, DH), lambda h, i: (h, i, 0)),
        out_shape=jax.ShapeDtypeStruct((H, S, DH), jnp.float32),
    )(q, k, v, cos, sin, cos, sin, sinks)


def _k3_body(attn_ref, wo_ref, bo_ref, res_ref, ln2_ref, rw_ref, rb_ref,
             h_ref, x2_ref, i1_ref, i2_ref, w1_ref, w2_ref):
    a = attn_ref[...]
    h = jnp.dot(a, wo_ref[...], preferred_element_type=jnp.float32) \
        + bo_ref[...] + res_ref[...]
    h_ref[...] = h
    x2 = _rms_in(h, ln2_ref[...])
    x2_ref[...] = x2
    rl = jnp.dot(x2, rw_ref[...], preferred_element_type=jnp.float32) + rb_ref[...]
    m1 = jnp.max(rl, axis=1, keepdims=True)
    ids = jax.lax.broadcasted_iota(jnp.int32, rl.shape, 1)
    i1 = jnp.min(jnp.where(rl == m1, ids, E), axis=1, keepdims=True)
    rl2 = jnp.where(ids == i1, -jnp.inf, rl)
    m2 = jnp.max(rl2, axis=1, keepdims=True)
    i2 = jnp.min(jnp.where(rl2 == m2, ids, E), axis=1, keepdims=True)
    b = jnp.exp(m2 - m1)
    w1 = 1.0 / (1.0 + b)
    i1_ref[...] = i1
    i2_ref[...] = i2
    w1_ref[...] = w1
    w2_ref[...] = b * w1


def _k3(attn, wo, bo, res, ln2_w, router_w, router_b):
    n = S // BS1
    return pl.pallas_call(
        _k3_body,
        grid=(n,),
        in_specs=[
            pl.BlockSpec((BS1, H * DH), lambda i: (i, 0)),
            pl.BlockSpec((H * DH, D), lambda i: (0, 0)),
            pl.BlockSpec((1, D), lambda i: (0, 0)),
            pl.BlockSpec((BS1, D), lambda i: (i, 0)),
            pl.BlockSpec((1, D), lambda i: (0, 0)),
            pl.BlockSpec((D, E), lambda i: (0, 0)),
            pl.BlockSpec((1, E), lambda i: (0, 0)),
        ],
        out_specs=[
            pl.BlockSpec((BS1, D), lambda i: (i, 0)),
            pl.BlockSpec((BS1, D), lambda i: (i, 0)),
            pl.BlockSpec((BS1, 1), lambda i: (i, 0)),
            pl.BlockSpec((BS1, 1), lambda i: (i, 0)),
            pl.BlockSpec((BS1, 1), lambda i: (i, 0)),
            pl.BlockSpec((BS1, 1), lambda i: (i, 0)),
        ],
        out_shape=[
            jax.ShapeDtypeStruct((S, D), jnp.float32),
            jax.ShapeDtypeStruct((S, D), jnp.float32),
            jax.ShapeDtypeStruct((S, 1), jnp.int32),
            jax.ShapeDtypeStruct((S, 1), jnp.int32),
            jax.ShapeDtypeStruct((S, 1), jnp.float32),
            jax.ShapeDtypeStruct((S, 1), jnp.float32),
        ],
    )(attn, wo, bo[None, :], res, ln2_w[None, :], router_w.T, router_b[None, :])


NCORE, NSUB, LANE = 2, 16, 16
NW = NCORE * NSUB
NPAD = S * TOPK + E * BT
NTILE = NPAD // BT
NGRP = 48


def _sc_mesh():
    return plsc.VectorSubcoreMesh(core_axis_name="c", subcore_axis_name="s",
                                  num_cores=NCORE, num_subcores=NSUB)


def _sc_route_body(i1_hbm, i2_hbm, st_hbm, p0_hbm, p1_hbm, grp_hbm,
                   i1_v, i2_v, st_v, p0_v, p1_v, cnt_v, seg_v, nxt_v, grp_v,
                   *, worker_id):
    @pl.when(worker_id == 0)
    def _():
        pltpu.sync_copy(i1_hbm, i1_v)
        pltpu.sync_copy(i2_hbm, i2_v)
        zeros16 = jnp.zeros((LANE,), jnp.int32)
        ones16 = jnp.full((LANE,), 1, jnp.int32)
        seven16 = jnp.full((LANE,), 7, jnp.int32)
        e7_16 = jnp.full((LANE,), E - 1, jnp.int32)
        bt16 = jnp.full((LANE,), BT, jnp.int32)
        lanes = lax.iota(jnp.int32, LANE)
        cnt_v[...] = zeros16

        def zi(i, c):
            st_v[pl.ds(i * LANE, LANE)] = zeros16
            return c
        lax.fori_loop(0, NPAD // LANE, zi, 0)

        def p1(i, c):
            plsc.addupdate_scatter(cnt_v, [i1_v[pl.ds(i * LANE, LANE)]], ones16)
            plsc.addupdate_scatter(cnt_v, [i2_v[pl.ds(i * LANE, LANE)]], ones16)
            return c
        lax.fori_loop(0, S // LANE, p1, 0)

        cnt = cnt_v[...]
        padded = ((cnt + jnp.full((LANE,), BT - 1, jnp.int32)) >> seven16) << seven16
        seg_end = plsc.cumsum(padded)
        plsc.store_scatter(seg_v, [lanes + jnp.full((LANE,), 8, jnp.int32)],
                           seg_end)
        nxt_v[...] = seg_end - padded

        for v in range(NGRP // LANE):
            jb = (lanes + jnp.full((LANE,), v * LANE, jnp.int32)) * bt16
            g = zeros16
            for e in range(E):
                se = plsc.load_gather(seg_v, [jnp.full((LANE,), 8 + e, jnp.int32)])
                g = g + (jb >= se).astype(jnp.int32)
            grp_v[pl.ds(v * LANE, LANE)] = jnp.minimum(g, e7_16)

        def p3(i, c):
            tokbase = lax.broadcast(i * LANE, (LANE,))
            for ids_v, pos_v in ((i1_v, p0_v), (i2_v, p1_v)):
                ids = ids_v[pl.ds(i * LANE, LANE)]
                base = plsc.load_gather(nxt_v, [ids])
                rank = zeros16
                for e in range(E):
                    m = ids == jnp.full((LANE,), e, jnp.int32)
                    mi = m.astype(jnp.int32)
                    rank = rank + jnp.where(m, plsc.cumsum(mi) - ones16, zeros16)
                pos = base + rank
                plsc.store_scatter(st_v, [pos], tokbase + lanes)
                pos_v[pl.ds(i * LANE, LANE)] = pos
                plsc.addupdate_scatter(nxt_v, [ids], ones16)
            return c
        lax.fori_loop(0, S // LANE, p3, 0)

        pltpu.sync_copy(st_v, st_hbm)
        pltpu.sync_copy(p0_v, p0_hbm)
        pltpu.sync_copy(p1_v, p1_hbm)
        pltpu.sync_copy(grp_v, grp_hbm)


def _sc_route(i1, i2):
    def body(i1_hbm, i2_hbm, st_hbm, p0_hbm, p1_hbm, grp_hbm,
             i1_v, i2_v, st_v, p0_v, p1_v, cnt_v, seg_v, nxt_v, grp_v):
        wid = lax.axis_index("s") * NCORE + lax.axis_index("c")
        _sc_route_body(i1_hbm, i2_hbm, st_hbm, p0_hbm, p1_hbm, grp_hbm,
                       i1_v, i2_v, st_v, p0_v, p1_v, cnt_v, seg_v, nxt_v,
                       grp_v, worker_id=wid)
    return pl.kernel(
        body,
        out_type=[
            jax.ShapeDtypeStruct((NPAD,), jnp.int32),
            jax.ShapeDtypeStruct((S,), jnp.int32),
            jax.ShapeDtypeStruct((S,), jnp.int32),
            jax.ShapeDtypeStruct((NGRP,), jnp.int32),
        ],
        mesh=_sc_mesh(),
        compiler_params=pltpu.CompilerParams(needs_layout_passes=False),
        scratch_types=[
            pltpu.VMEM((S,), jnp.int32),
            pltpu.VMEM((S,), jnp.int32),
            pltpu.VMEM((NPAD,), jnp.int32),
            pltpu.VMEM((S,), jnp.int32),
            pltpu.VMEM((S,), jnp.int32),
            pltpu.VMEM((LANE,), jnp.int32),
            pltpu.VMEM((2 * LANE,), jnp.int32),
            pltpu.VMEM((LANE,), jnp.int32),
            pltpu.VMEM((NGRP,), jnp.int32),
        ],
    )(i1, i2)


GCH = 32


def _sc_gather_body(st_hbm, x2_hbm, xs_hbm, idx_v, rows_v, sem):
    wid = lax.axis_index("s") * NCORE + lax.axis_index("c")
    per_w = NPAD // NW

    def chunk(c, carry):
        off = pl.multiple_of(wid * per_w + c * GCH, GCH)
        pltpu.sync_copy(st_hbm.at[pl.ds(off, GCH)], idx_v)
        pltpu.async_copy(x2_hbm.at[idx_v], rows_v, sem).wait()
        pltpu.sync_copy(rows_v, xs_hbm.at[pl.ds(off, GCH)])
        return carry
    lax.fori_loop(0, per_w // GCH, chunk, 0)


def _sc_gather(st, x2):
    return pl.kernel(
        _sc_gather_body,
        out_type=jax.ShapeDtypeStruct((NPAD, D), jnp.float32),
        mesh=_sc_mesh(),
        compiler_params=pltpu.CompilerParams(needs_layout_passes=False),
        scratch_types=[
            pltpu.VMEM((GCH,), jnp.int32),
            pltpu.VMEM((GCH, D), jnp.float32),
            pltpu.SemaphoreType.DMA,
        ],
    )(st, x2)


def _sc_combine_body(p0_hbm, p1_hbm, os_hbm, g0_hbm, g1_hbm, idx_v, rows_v, sem):
    wid = lax.axis_index("s") * NCORE + lax.axis_index("c")
    per_w = S // NW

    def chunk(c, carry):
        off = pl.multiple_of(wid * per_w + c * GCH, GCH)
        for p_hbm, g_hbm in ((p0_hbm, g0_hbm), (p1_hbm, g1_hbm)):
            pltpu.sync_copy(p_hbm.at[pl.ds(off, GCH)], idx_v)
            pltpu.async_copy(os_hbm.at[idx_v], rows_v, sem).wait()
            pltpu.sync_copy(rows_v, g_hbm.at[pl.ds(off, GCH)])
        return carry
    lax.fori_loop(0, per_w // GCH, chunk, 0)


def _sc_combine(p0, p1, os):
    return pl.kernel(
        _sc_combine_body,
        out_type=[
            jax.ShapeDtypeStruct((S, D), jnp.float32),
            jax.ShapeDtypeStruct((S, D), jnp.float32),
        ],
        mesh=_sc_mesh(),
        compiler_params=pltpu.CompilerParams(needs_layout_passes=False),
        scratch_types=[
            pltpu.VMEM((GCH,), jnp.int32),
            pltpu.VMEM((GCH, D), jnp.float32),
            pltpu.SemaphoreType.DMA,
        ],
    )(p0, p1, os)


def _k4_body(g_ref, xs_ref, wgu_ref, bgu_ref, pg_ref, wd_ref, bd_ref, o_ref):
    x = xs_ref[...]
    gu = jnp.dot(x, wgu_ref[0], preferred_element_type=jnp.float32) + bgu_ref[0]
    gate_v = jnp.minimum(gu, LIMIT)
    up_v = jnp.clip(pltpu.roll(gu, 2 * FF - 1, 1), -LIMIT, LIMIT)
    glu_v = gate_v * jax.nn.sigmoid(gate_v * ALPHA)
    act_v = (up_v + 1.0) * glu_v
    act = jnp.dot(act_v, pg_ref[...], preferred_element_type=jnp.float32)
    o_ref[...] = jnp.dot(act, wd_ref[0], preferred_element_type=jnp.float32) \
        + bd_ref[0]


def _k4(grp, xs, wgu, bgu, pg, wd, bd):
    grid_spec = pltpu.PrefetchScalarGridSpec(
        num_scalar_prefetch=1,
        grid=(NTILE,),
        in_specs=[
            pl.BlockSpec((BT, D), lambda j, g: (j, 0)),
            pl.BlockSpec((1, D, 2 * FF), lambda j, g: (g[j], 0, 0)),
            pl.BlockSpec((1, 1, 2 * FF), lambda j, g: (g[j], 0, 0)),
            pl.BlockSpec((2 * FF, FF), lambda j, g: (0, 0)),
            pl.BlockSpec((1, FF, D), lambda j, g: (g[j], 0, 0)),
            pl.BlockSpec((1, 1, D), lambda j, g: (g[j], 0, 0)),
        ],
        out_specs=pl.BlockSpec((BT, D), lambda j, g: (j, 0)),
    )
    return pl.pallas_call(
        _k4_body,
        grid_spec=grid_spec,
        out_shape=jax.ShapeDtypeStruct((NPAD, D), jnp.float32),
        compiler_params=pltpu.CompilerParams(
            dimension_semantics=("arbitrary",),
        ),
    )(grp, xs, wgu, bgu, pg, wd, bd)


def _k5_body(h_ref, g0_ref, g1_ref, w1_ref, w2_ref, o_ref):
    o_ref[...] = h_ref[...] + w1_ref[...] * g0_ref[...] \
        + w2_ref[...] * g1_ref[...]


def _k5(h1, g0, g1, w1, w2):
    n = S // BS1
    return pl.pallas_call(
        _k5_body,
        grid=(n,),
        in_specs=[
            pl.BlockSpec((BS1, D), lambda i: (i, 0)),
            pl.BlockSpec((BS1, D), lambda i: (i, 0)),
            pl.BlockSpec((BS1, D), lambda i: (i, 0)),
            pl.BlockSpec((BS1, 1), lambda i: (i, 0)),
            pl.BlockSpec((BS1, 1), lambda i: (i, 0)),
        ],
        out_specs=pl.BlockSpec((BS1, D), lambda i: (i, 0)),
        out_shape=jax.ShapeDtypeStruct((S, D), jnp.float32),
    )(h1, g0, g1, w1, w2)


def kernel(hidden_states, ln1_w, wq, bq, wk, bk, wv, bv, wo, bo, sinks, ln2_w,
           router_w, router_b, gate_up_proj, gate_up_bias, down_proj, down_bias):
    h0 = hidden_states.reshape(S, D)
    q, k, v = _k1(h0, ln1_w, wq, bq, wk, bk, wv, bv)
    qh = q.reshape(S, H, DH).transpose(1, 0, 2)
    kh = k.reshape(S, KV, DH).transpose(1, 0, 2)
    vh = v.reshape(S, KV, DH).transpose(1, 0, 2)
    inv = 1.0 / (THETA ** (jnp.arange(0, DH, 2, dtype=jnp.float32) / DH))
    t = jnp.arange(S, dtype=jnp.float32)
    f = jnp.outer(t, inv)
    cos, sin = jnp.cos(f), jnp.sin(f)
    oh = _k2(qh, kh, vh, cos, sin, sinks)
    attn = oh.transpose(1, 0, 2).reshape(S, H * DH)
    h1, x2, i1, i2, w1, w2 = _k3(attn, wo, bo, h0, ln2_w, router_w, router_b)
    st, p0, p1, grp = _sc_route(i1.reshape(S), i2.reshape(S))
    xs = _sc_gather(st, x2)
    pg = jnp.equal(jnp.arange(2 * FF)[:, None], 2 * jnp.arange(FF)[None, :]
                   ).astype(jnp.float32)
    os_ = _k4(grp, xs, gate_up_proj, gate_up_bias[:, None, :], pg, down_proj,
              down_bias[:, None, :])
    g0, g1 = _sc_combine(p0, p1, os_)
    out = _k5(h1, g0, g1, w1, w2)
    return out.reshape(B, S, D)


# speedup vs baseline: 1.0960x; 1.0960x over previous
import functools
import math

import jax
import jax.numpy as jnp
from jax import lax
from jax.experimental import pallas as pl
from jax.experimental.pallas import tpu as pltpu
from jax.experimental.pallas import tpu_sc as plsc

B, S, D = 1, 2048, 768
H, KV, DH = 12, 4, 64
E, TOPK, FF = 8, 2, 768
EPS = 1e-5
THETA = 150000.0
ALPHA = 1.702
LIMIT = 7.0

BS1 = 256
BQ = 256
BT = 128


def _rms_in(x, w):
    v = jnp.mean(jnp.square(x), axis=-1, keepdims=True)
    return x * jax.lax.rsqrt(v + EPS) * w


def _k1_body(h_ref, ln1_ref, wq_ref, bq_ref, wk_ref, bk_ref, wv_ref, bv_ref,
             q_ref, k_ref, v_ref):
    x = _rms_in(h_ref[...], ln1_ref[...])
    q_ref[...] = jnp.dot(x, wq_ref[...], preferred_element_type=jnp.float32) + bq_ref[...]
    k_ref[...] = jnp.dot(x, wk_ref[...], preferred_element_type=jnp.float32) + bk_ref[...]
    v_ref[...] = jnp.dot(x, wv_ref[...], preferred_element_type=jnp.float32) + bv_ref[...]


def _k1(h, ln1_w, wq, bq, wk, bk, wv, bv):
    n = S // BS1
    return pl.pallas_call(
        _k1_body,
        grid=(n,),
        in_specs=[
            pl.BlockSpec((BS1, D), lambda i: (i, 0)),
            pl.BlockSpec((1, D), lambda i: (0, 0)),
            pl.BlockSpec((D, H * DH), lambda i: (0, 0)),
            pl.BlockSpec((1, H * DH), lambda i: (0, 0)),
            pl.BlockSpec((D, KV * DH), lambda i: (0, 0)),
            pl.BlockSpec((1, KV * DH), lambda i: (0, 0)),
            pl.BlockSpec((D, KV * DH), lambda i: (0, 0)),
            pl.BlockSpec((1, KV * DH), lambda i: (0, 0)),
        ],
        out_specs=[
            pl.BlockSpec((BS1, H * DH), lambda i: (i, 0)),
            pl.BlockSpec((BS1, KV * DH), lambda i: (i, 0)),
            pl.BlockSpec((BS1, KV * DH), lambda i: (i, 0)),
        ],
        out_shape=[
            jax.ShapeDtypeStruct((S, H * DH), jnp.float32),
            jax.ShapeDtypeStruct((S, KV * DH), jnp.float32),
            jax.ShapeDtypeStruct((S, KV * DH), jnp.float32),
        ],
    )(h, ln1_w[None, :], wq, bq[None, :], wk, bk[None, :], wv, bv[None, :])


def _rope(x, cos, sin):
    x1 = x[:, :DH // 2]
    x2 = x[:, DH // 2:]
    return jnp.concatenate([x1 * cos - x2 * sin, x2 * cos + x1 * sin], axis=1)


def _k2_body(q_ref, k_ref, v_ref, cq_ref, sq_ref, ck_ref, sk_ref, sink_ref,
             o_ref):
    h = pl.program_id(0)
    iq = pl.program_id(1)
    q = _rope(q_ref[0], cq_ref[...], sq_ref[...]) * (1.0 / math.sqrt(DH))
    k = _rope(k_ref[0], ck_ref[...], sk_ref[...])
    logits = jax.lax.dot_general(q, k, (((1,), (1,)), ((), ())),
                                 preferred_element_type=jnp.float32)
    row = iq * BQ + jax.lax.broadcasted_iota(jnp.int32, (BQ, S), 0)
    col = jax.lax.broadcasted_iota(jnp.int32, (BQ, S), 1)
    logits = jnp.where(col <= row, logits, -1e9)
    sink = sink_ref[h]
    m = jnp.maximum(jnp.max(logits, axis=1, keepdims=True), sink)
    p = jnp.exp(logits - m)
    denom = jnp.sum(p, axis=1, keepdims=True) + jnp.exp(sink - m)
    o = jnp.dot(p, v_ref[0], preferred_element_type=jnp.float32)
    o_ref[0] = o / denom


def _k2(q, k, v, cos, sin, sinks):
    rep = H // KV
    nq = S // BQ
    return pl.pallas_call(
        _k2_body,
        grid=(H, nq),
        in_specs=[
            pl.BlockSpec((1, BQ, DH), lambda h, i: (h, i, 0)),
            pl.BlockSpec((1, S, DH), lambda h, i: (h // rep, 0, 0)),
            pl.BlockSpec((1, S, DH), lambda h, i: (h // rep, 0, 0)),
            pl.BlockSpec((BQ, DH // 2), lambda h, i: (i, 0)),
            pl.BlockSpec((BQ, DH // 2), lambda h, i: (i, 0)),
            pl.BlockSpec((S, DH // 2), lambda h, i: (0, 0)),
            pl.BlockSpec((S, DH // 2), lambda h, i: (0, 0)),
            pl.BlockSpec(memory_space=pltpu.SMEM),
        ],
        out_specs=pl.BlockSpec((1, BQ, DH), lambda h, i: (h, i, 0)),
        out_shape=jax.ShapeDtypeStruct((H, S, DH), jnp.float32),
    )(q, k, v, cos, sin, cos, sin, sinks)


def _k3_body(attn_ref, wo_ref, bo_ref, res_ref, ln2_ref, rw_ref, rb_ref,
             h_ref, x2_ref, i1_ref, i2_ref, w1_ref, w2_ref):
    a = attn_ref[...]
    h = jnp.dot(a, wo_ref[...], preferred_element_type=jnp.float32) \
        + bo_ref[...] + res_ref[...]
    h_ref[...] = h
    x2 = _rms_in(h, ln2_ref[...])
    x2_ref[...] = x2
    rl = jnp.dot(x2, rw_ref[...], preferred_element_type=jnp.float32) + rb_ref[...]
    m1 = jnp.max(rl, axis=1, keepdims=True)
    ids = jax.lax.broadcasted_iota(jnp.int32, rl.shape, 1)
    i1 = jnp.min(jnp.where(rl == m1, ids, E), axis=1, keepdims=True)
    rl2 = jnp.where(ids == i1, -jnp.inf, rl)
    m2 = jnp.max(rl2, axis=1, keepdims=True)
    i2 = jnp.min(jnp.where(rl2 == m2, ids, E), axis=1, keepdims=True)
    b = jnp.exp(m2 - m1)
    w1 = 1.0 / (1.0 + b)
    i1_ref[...] = i1
    i2_ref[...] = i2
    w1_ref[...] = w1
    w2_ref[...] = b * w1


def _k3(attn, wo, bo, res, ln2_w, router_w, router_b):
    n = S // BS1
    return pl.pallas_call(
        _k3_body,
        grid=(n,),
        in_specs=[
            pl.BlockSpec((BS1, H * DH), lambda i: (i, 0)),
            pl.BlockSpec((H * DH, D), lambda i: (0, 0)),
            pl.BlockSpec((1, D), lambda i: (0, 0)),
            pl.BlockSpec((BS1, D), lambda i: (i, 0)),
            pl.BlockSpec((1, D), lambda i: (0, 0)),
            pl.BlockSpec((D, E), lambda i: (0, 0)),
            pl.BlockSpec((1, E), lambda i: (0, 0)),
        ],
        out_specs=[
            pl.BlockSpec((BS1, D), lambda i: (i, 0)),
            pl.BlockSpec((BS1, D), lambda i: (i, 0)),
            pl.BlockSpec((BS1, 1), lambda i: (i, 0)),
            pl.BlockSpec((BS1, 1), lambda i: (i, 0)),
            pl.BlockSpec((BS1, 1), lambda i: (i, 0)),
            pl.BlockSpec((BS1, 1), lambda i: (i, 0)),
        ],
        out_shape=[
            jax.ShapeDtypeStruct((S, D), jnp.float32),
            jax.ShapeDtypeStruct((S, D), jnp.float32),
            jax.ShapeDtypeStruct((S, 1), jnp.int32),
            jax.ShapeDtypeStruct((S, 1), jnp.int32),
            jax.ShapeDtypeStruct((S, 1), jnp.float32),
            jax.ShapeDtypeStruct((S, 1), jnp.float32),
        ],
    )(attn, wo, bo[None, :], res, ln2_w[None, :], router_w.T, router_b[None, :])


NCORE, NSUB, LANE = 2, 16, 16
NW = NCORE * NSUB
NPAD = S * TOPK + E * BT
NTILE = NPAD // BT
NGRP = 48


def _sc_mesh():
    return plsc.VectorSubcoreMesh(core_axis_name="c", subcore_axis_name="s",
                                  num_cores=NCORE, num_subcores=NSUB)


def _sc_route_body(i1_hbm, i2_hbm, st_hbm, p0_hbm, p1_hbm, grp_hbm,
                   i1_v, i2_v, st_v, p0_v, p1_v, cnt_v, seg_v, nxt_v, grp_v,
                   *, worker_id):
    @pl.when(worker_id == 0)
    def _():
        pltpu.sync_copy(i1_hbm, i1_v)
        pltpu.sync_copy(i2_hbm, i2_v)
        zeros16 = jnp.zeros((LANE,), jnp.int32)
        ones16 = jnp.full((LANE,), 1, jnp.int32)
        seven16 = jnp.full((LANE,), 7, jnp.int32)
        e7_16 = jnp.full((LANE,), E - 1, jnp.int32)
        bt16 = jnp.full((LANE,), BT, jnp.int32)
        lanes = lax.iota(jnp.int32, LANE)
        cnt_v[...] = zeros16

        def zi(i, c):
            st_v[pl.ds(i * LANE, LANE)] = zeros16
            return c
        lax.fori_loop(0, NPAD // LANE, zi, 0)

        def p1(i, c):
            plsc.addupdate_scatter(cnt_v, [i1_v[pl.ds(i * LANE, LANE)]], ones16)
            plsc.addupdate_scatter(cnt_v, [i2_v[pl.ds(i * LANE, LANE)]], ones16)
            return c
        lax.fori_loop(0, S // LANE, p1, 0)

        cnt = cnt_v[...]
        padded = ((cnt + jnp.full((LANE,), BT - 1, jnp.int32)) >> seven16) << seven16
        seg_end = plsc.cumsum(padded)
        plsc.store_scatter(seg_v, [lanes + jnp.full((LANE,), 8, jnp.int32)],
                           seg_end)
        nxt_v[...] = seg_end - padded

        for v in range(NGRP // LANE):
            jb = (lanes + jnp.full((LANE,), v * LANE, jnp.int32)) * bt16
            g = zeros16
            for e in range(E):
                se = plsc.load_gather(seg_v, [jnp.full((LANE,), 8 + e, jnp.int32)])
                g = g + (jb >= se).astype(jnp.int32)
            grp_v[pl.ds(v * LANE, LANE)] = jnp.minimum(g, e7_16)

        def p3(i, c):
            tokbase = lax.broadcast(i * LANE, (LANE,))
            for ids_v, pos_v in ((i1_v, p0_v), (i2_v, p1_v)):
                ids = ids_v[pl.ds(i * LANE, LANE)]
                base = plsc.load_gather(nxt_v, [ids])
                rank = zeros16
                for e in range(E):
                    m = ids == jnp.full((LANE,), e, jnp.int32)
                    mi = m.astype(jnp.int32)
                    rank = rank + jnp.where(m, plsc.cumsum(mi) - ones16, zeros16)
                pos = base + rank
                plsc.store_scatter(st_v, [pos], tokbase + lanes)
                pos_v[pl.ds(i * LANE, LANE)] = pos
                plsc.addupdate_scatter(nxt_v, [ids], ones16)
            return c
        lax.fori_loop(0, S // LANE, p3, 0)

        pltpu.sync_copy(st_v, st_hbm)
        pltpu.sync_copy(p0_v, p0_hbm)
        pltpu.sync_copy(p1_v, p1_hbm)
        pltpu.sync_copy(grp_v, grp_hbm)


def _sc_route(i1, i2):
    def body(i1_hbm, i2_hbm, st_hbm, p0_hbm, p1_hbm, grp_hbm,
             i1_v, i2_v, st_v, p0_v, p1_v, cnt_v, seg_v, nxt_v, grp_v):
        wid = lax.axis_index("s") * NCORE + lax.axis_index("c")
        _sc_route_body(i1_hbm, i2_hbm, st_hbm, p0_hbm, p1_hbm, grp_hbm,
                       i1_v, i2_v, st_v, p0_v, p1_v, cnt_v, seg_v, nxt_v,
                       grp_v, worker_id=wid)
    return pl.kernel(
        body,
        out_type=[
            jax.ShapeDtypeStruct((NPAD,), jnp.int32),
            jax.ShapeDtypeStruct((S,), jnp.int32),
            jax.ShapeDtypeStruct((S,), jnp.int32),
            jax.ShapeDtypeStruct((NGRP,), jnp.int32),
        ],
        mesh=_sc_mesh(),
        compiler_params=pltpu.CompilerParams(needs_layout_passes=False),
        scratch_types=[
            pltpu.VMEM((S,), jnp.int32),
            pltpu.VMEM((S,), jnp.int32),
            pltpu.VMEM((NPAD,), jnp.int32),
            pltpu.VMEM((S,), jnp.int32),
            pltpu.VMEM((S,), jnp.int32),
            pltpu.VMEM((LANE,), jnp.int32),
            pltpu.VMEM((2 * LANE,), jnp.int32),
            pltpu.VMEM((LANE,), jnp.int32),
            pltpu.VMEM((NGRP,), jnp.int32),
        ],
    )(i1, i2)


GCH = 32


def _sc_gather_body(st_hbm, x2_hbm, xs_hbm, idx_v, rows_v, sem):
    wid = lax.axis_index("s") * NCORE + lax.axis_index("c")
    per_w = NPAD // NW

    def chunk(c, carry):
        off = pl.multiple_of(wid * per_w + c * GCH, GCH)
        pltpu.sync_copy(st_hbm.at[pl.ds(off, GCH)], idx_v)
        pltpu.async_copy(x2_hbm.at[idx_v], rows_v, sem).wait()
        pltpu.sync_copy(rows_v, xs_hbm.at[pl.ds(off, GCH)])
        return carry
    lax.fori_loop(0, per_w // GCH, chunk, 0)


def _sc_gather(st, x2):
    return pl.kernel(
        _sc_gather_body,
        out_type=jax.ShapeDtypeStruct((NPAD, D), jnp.float32),
        mesh=_sc_mesh(),
        compiler_params=pltpu.CompilerParams(needs_layout_passes=False),
        scratch_types=[
            pltpu.VMEM((GCH,), jnp.int32),
            pltpu.VMEM((GCH, D), jnp.float32),
            pltpu.SemaphoreType.DMA,
        ],
    )(st, x2)


def _sc_combine_body(p0_hbm, p1_hbm, os_hbm, g0_hbm, g1_hbm, idx_v, rows_v, sem):
    wid = lax.axis_index("s") * NCORE + lax.axis_index("c")
    per_w = S // NW

    def chunk(c, carry):
        off = pl.multiple_of(wid * per_w + c * GCH, GCH)
        for p_hbm, g_hbm in ((p0_hbm, g0_hbm), (p1_hbm, g1_hbm)):
            pltpu.sync_copy(p_hbm.at[pl.ds(off, GCH)], idx_v)
            pltpu.async_copy(os_hbm.at[idx_v], rows_v, sem).wait()
            pltpu.sync_copy(rows_v, g_hbm.at[pl.ds(off, GCH)])
        return carry
    lax.fori_loop(0, per_w // GCH, chunk, 0)


def _sc_combine(p0, p1, os):
    return pl.kernel(
        _sc_combine_body,
        out_type=[
            jax.ShapeDtypeStruct((S, D), jnp.float32),
            jax.ShapeDtypeStruct((S, D), jnp.float32),
        ],
        mesh=_sc_mesh(),
        compiler_params=pltpu.CompilerParams(needs_layout_passes=False),
        scratch_types=[
            pltpu.VMEM((GCH,), jnp.int32),
            pltpu.VMEM((GCH, D), jnp.float32),
            pltpu.SemaphoreType.DMA,
        ],
    )(p0, p1, os)


def _k4_body(g_ref, xs_ref, wgu_ref, bgu_ref, pg_ref, wd_ref, bd_ref, o_ref):
    x = xs_ref[...]
    gu = jnp.dot(x, wgu_ref[0], preferred_element_type=jnp.float32) + bgu_ref[0]
    gate_v = jnp.minimum(gu, LIMIT)
    up_v = jnp.clip(pltpu.roll(gu, 2 * FF - 1, 1), -LIMIT, LIMIT)
    glu_v = gate_v * jax.nn.sigmoid(gate_v * ALPHA)
    act_v = (up_v + 1.0) * glu_v
    act = jnp.dot(act_v, pg_ref[...], preferred_element_type=jnp.float32)
    o_ref[...] = jnp.dot(act, wd_ref[0], preferred_element_type=jnp.float32) \
        + bd_ref[0]


def _k4(grp, xs, wgu, bgu, pg, wd, bd):
    grid_spec = pltpu.PrefetchScalarGridSpec(
        num_scalar_prefetch=1,
        grid=(NTILE,),
        in_specs=[
            pl.BlockSpec((BT, D), lambda j, g: (j, 0)),
            pl.BlockSpec((1, D, 2 * FF), lambda j, g: (g[j], 0, 0)),
            pl.BlockSpec((1, 1, 2 * FF), lambda j, g: (g[j], 0, 0)),
            pl.BlockSpec((2 * FF, FF), lambda j, g: (0, 0)),
            pl.BlockSpec((1, FF, D), lambda j, g: (g[j], 0, 0)),
            pl.BlockSpec((1, 1, D), lambda j, g: (g[j], 0, 0)),
        ],
        out_specs=pl.BlockSpec((BT, D), lambda j, g: (j, 0)),
    )
    return pl.pallas_call(
        _k4_body,
        grid_spec=grid_spec,
        out_shape=jax.ShapeDtypeStruct((NPAD, D), jnp.float32),
        compiler_params=pltpu.CompilerParams(
            dimension_semantics=("arbitrary",),
        ),
    )(grp, xs, wgu, bgu, pg, wd, bd)


def _k5_body(h_ref, g0_ref, g1_ref, w1_ref, w2_ref, o_ref):
    o_ref[...] = h_ref[...] + w1_ref[...] * g0_ref[...] \
        + w2_ref[...] * g1_ref[...]


def _k5(h1, g0, g1, w1, w2):
    n = S // BS1
    return pl.pallas_call(
        _k5_body,
        grid=(n,),
        in_specs=[
            pl.BlockSpec((BS1, D), lambda i: (i, 0)),
            pl.BlockSpec((BS1, D), lambda i: (i, 0)),
            pl.BlockSpec((BS1, D), lambda i: (i, 0)),
            pl.BlockSpec((BS1, 1), lambda i: (i, 0)),
            pl.BlockSpec((BS1, 1), lambda i: (i, 0)),
        ],
        out_specs=pl.BlockSpec((BS1, D), lambda i: (i, 0)),
        out_shape=jax.ShapeDtypeStruct((S, D), jnp.float32),
    )(h1, g0, g1, w1, w2)


def kernel(hidden_states, ln1_w, wq, bq, wk, bk, wv, bv, wo, bo, sinks, ln2_w,
           router_w, router_b, gate_up_proj, gate_up_bias, down_proj, down_bias):
    h0 = hidden_states.reshape(S, D)
    q, k, v = _k1(h0, ln1_w, wq, bq, wk, bk, wv, bv)
    qh = q.reshape(S, H, DH).transpose(1, 0, 2)
    kh = k.reshape(S, KV, DH).transpose(1, 0, 2)
    vh = v.reshape(S, KV, DH).transpose(1, 0, 2)
    inv = 1.0 / (THETA ** (jnp.arange(0, DH, 2, dtype=jnp.float32) / DH))
    t = jnp.arange(S, dtype=jnp.float32)
    f = jnp.outer(t, inv)
    cos, sin = jnp.cos(f), jnp.sin(f)
    oh = _k2(qh, kh, vh, cos, sin, sinks)
    attn = oh.transpose(1, 0, 2).reshape(S, H * DH)
    h1, x2, i1, i2, w1, w2 = _k3(attn, wo, bo, h0, ln2_w, router_w, router_b)
    st, p0, p1, grp = _sc_route(i1.reshape(S), i2.reshape(S))
    xs = _sc_gather(st, x2)
    pg = jnp.equal(jnp.arange(2 * FF)[:, None], 2 * jnp.arange(FF)[None, :]
                   ).astype(jnp.float32)
    os_ = _k4(grp, xs, gate_up_proj, gate_up_bias[:, None, :], pg, down_proj,
              down_bias[:, None, :])
    g0, g1 = _sc_combine(p0, p1, os_)
    out = _k5(h1, g0, g1, w1, w2)
    return out.reshape(B, S, D)
